# baseline (speedup 1.0000x reference)
import functools

import jax
import jax.numpy as jnp
from jax import lax
from jax.experimental import pallas as pl
from jax.experimental.pallas import tpu as pltpu
from jax.experimental.pallas import tpu_sc as plsc

BATCH = 16384
EMBED_DIM = 64
NROWS = 1000000
LANES = 128
NCHUNK = 7813
TAIL_C = 7812
TAIL_OFF = TAIL_C * LANES
TAIL_W = NROWS - TAIL_OFF
CAP = 640
MAXCH = 245
MAXCHP = 248
CAPP = CAP + 16

_info = plsc.get_sparse_core_info()
_NC, _NS, _L = _info.num_cores, _info.num_subcores, _info.num_lanes
_NW = _NC * _NS
_BPW = BATCH // _NW
_IROWS = _NW * CAP // 2


def _extract_body(utabT_hbm, mtabT_hbm, utailT_hbm, mtailT_hbm,
                  ulane_hbm, uch_hbm, mlane_hbm, mch_hbm,
                  interu_hbm, interm_hbm,
                  ulane_v, uch_v, mlane_v, mch_v,
                  ubuf, mbuf, stage_u, stage_m, sem_u, sem_m):
    w = lax.axis_index("s") * _NC + lax.axis_index("c")
    ch0 = (w * NCHUNK) >> 5
    ch1 = ((w + 1) * NCHUNK) >> 5
    nch = ch1 - ch0

    pltpu.sync_copy(ulane_hbm.at[pl.ds(w * CAPP, CAPP)], ulane_v)
    pltpu.sync_copy(uch_hbm.at[pl.ds(w * CAPP, CAPP)], uch_v)
    pltpu.sync_copy(mlane_hbm.at[pl.ds(w * CAPP, CAPP)], mlane_v)
    pltpu.sync_copy(mch_hbm.at[pl.ds(w * CAPP, CAPP)], mch_v)

    def issue(ci, s):
        c = ch0 + ci

        @pl.when(c < TAIL_C)
        def _():
            pltpu.async_copy(utabT_hbm.at[:, pl.ds(c * LANES, LANES)],
                             ubuf.at[s], sem_u.at[s])
            pltpu.async_copy(mtabT_hbm.at[:, pl.ds(c * LANES, LANES)],
                             mbuf.at[s], sem_m.at[s])

        @pl.when(c == TAIL_C)
        def _():
            pltpu.async_copy(utailT_hbm, ubuf.at[s], sem_u.at[s])
            pltpu.async_copy(mtailT_hbm, mbuf.at[s], sem_m.at[s])

    def wait(s):
        pltpu.make_async_copy(utabT_hbm.at[:, pl.ds(0, LANES)],
                              ubuf.at[s], sem_u.at[s]).wait()
        pltpu.make_async_copy(mtabT_hbm.at[:, pl.ds(0, LANES)],
                              mbuf.at[s], sem_m.at[s]).wait()

    iota = lax.iota(jnp.int32, _L)

    def extract(buf_slot, lane_v, ch_v, stage, p, c):
        def do_window(p0):
            lanes = plsc.load_gather(lane_v, [p0 + iota])
            chs = plsc.load_gather(ch_v, [p0 + iota])
            mask = chs == c
            cnt = plsc.all_reduce_population_count(mask)[0]
            slots = p0 + iota
            base = lax.shift_right_logical(slots, 1) * (2 * EMBED_DIM) \
                + (slots & 1) * EMBED_DIM
            for d in range(EMBED_DIM):
                vals = plsc.load_gather(
                    buf_slot, [jnp.full((_L,), d, jnp.int32), lanes])
                plsc.store_scatter(stage, [base + d], vals, mask=mask)
            return cnt

        cnt0 = do_window(p)
        p1 = p + cnt0

        @pl.when(cnt0 >= _L)
        def _():
            do_window(p1)

        chs1 = plsc.load_gather(ch_v, [p1 + iota])
        cnt1 = plsc.all_reduce_population_count(chs1 == c)[0]
        return p1 + jnp.where(cnt0 >= _L, cnt1, 0)

    issue(0, 0)

    def group_body(g, carry):
        p_u, p_m = carry
        for s in range(2):
            ci = g * 2 + s
            c = ch0 + ci

            @pl.when(ci < nch)
            def _():
                wait(s)

            @pl.when(ci + 1 < nch)
            def _():
                issue(ci + 1, 1 - s)

            p_u = extract(ubuf.at[s], ulane_v, uch_v, stage_u, p_u, c)
            p_m = extract(mbuf.at[s], mlane_v, mch_v, stage_m, p_m, c)
        return (p_u, p_m)

    lax.fori_loop(0, (MAXCH + 1) // 2, group_body,
                  (jnp.int32(0), jnp.int32(0)))

    pltpu.sync_copy(stage_u,
                    interu_hbm.at[pl.ds(w * (CAP * EMBED_DIM),
                                        CAP * EMBED_DIM)])
    pltpu.sync_copy(stage_m,
                    interm_hbm.at[pl.ds(w * (CAP * EMBED_DIM),
                                        CAP * EMBED_DIM)])


_PASS = 2
_RPP = _BPW // _PASS
_CPP = _RPP // _L


def _dot_body(urank_hbm, mrank_hbm, interu_hbm, interm_hbm, out_hbm,
              urank_v, mrank_v, ublk_v, mblk_v, ubi_v, mbi_v, out_v,
              sem_u, sem_m):
    wid = lax.axis_index("s") * _NC + lax.axis_index("c")
    base = wid * _BPW

    pltpu.sync_copy(urank_hbm.at[pl.ds(base, _BPW)], urank_v)
    pltpu.sync_copy(mrank_hbm.at[pl.ds(base, _BPW)], mrank_v)

    def do_pass(p, carry):
        def bi_body(i, c2):
            s = i * _L
            u = urank_v[pl.ds(p * _RPP + s, _L)]
            m = mrank_v[pl.ds(p * _RPP + s, _L)]
            ubi_v[pl.ds(s, _L)] = lax.shift_right_logical(u, 1)
            mbi_v[pl.ds(s, _L)] = lax.shift_right_logical(m, 1)
            return c2

        lax.fori_loop(0, _CPP, bi_body, 0)

        cu = pltpu.async_copy(interu_hbm.at[ubi_v], ublk_v, sem_u)
        cm = pltpu.async_copy(interm_hbm.at[mbi_v], mblk_v, sem_m)
        cu.wait()
        cm.wait()

        def chunk_body(c, c2):
            row_idx = c * _L + lax.iota(jnp.int32, _L)
            uo = urank_v[pl.ds(p * _RPP + c * _L, _L)]
            mo = mrank_v[pl.ds(p * _RPP + c * _L, _L)]
            ucol0 = (uo & 1) * EMBED_DIM
            mcol0 = (mo & 1) * EMBED_DIM
            acc = jnp.zeros((_L,), jnp.float32)
            for d in range(EMBED_DIM):
                u = plsc.load_gather(ublk_v, [row_idx, ucol0 + d])
                m = plsc.load_gather(mblk_v, [row_idx, mcol0 + d])
                acc = acc + u * m
            out_v[pl.ds(p * _RPP + c * _L, _L)] = acc
            return c2

        lax.fori_loop(0, _CPP, chunk_body, 0)
        return carry

    lax.fori_loop(0, _PASS, do_pass, 0)
    pltpu.sync_copy(out_v, out_hbm.at[pl.ds(base, _BPW)])


def _route(r):
    n = r.shape[0]
    order = jnp.argsort(r)
    rs = r[order]
    chb = ((jnp.arange(_NW + 1, dtype=jnp.int32) * NCHUNK) // _NW)
    st = jnp.searchsorted(rs, (chb * LANES).astype(rs.dtype)).astype(jnp.int32)
    k = jnp.arange(n, dtype=jnp.int32)
    w_k = jnp.clip(
        jnp.searchsorted(st, k, side="right").astype(jnp.int32) - 1, 0, _NW - 1)
    pos = k - st[w_k]
    lanes = jnp.zeros((_NW, CAPP), jnp.int32).at[w_k, pos].set(
        rs & (LANES - 1), mode="drop")
    chs = jnp.full((_NW, CAPP), -1, jnp.int32).at[w_k, pos].set(
        lax.shift_right_logical(rs, 7), mode="drop")
    rank = jnp.zeros((n,), jnp.int32).at[order].set(w_k * CAP + pos)
    return lanes.reshape(-1), chs.reshape(-1), rank


def kernel(user_indices, movie_indices, user_table, movie_table):
    uidx = user_indices.astype(jnp.int32)
    midx = movie_indices.astype(jnp.int32)
    utabT = user_table.T
    mtabT = movie_table.T
    ulane, uch, urank = _route(uidx)
    mlane, mch, mrank = _route(midx)

    mesh = plsc.VectorSubcoreMesh(core_axis_name="c", subcore_axis_name="s")
    params = pltpu.CompilerParams(
        needs_layout_passes=False, use_tc_tiling_on_sc=True)

    extract = functools.partial(
        pl.kernel,
        mesh=mesh,
        out_type=(
            jax.ShapeDtypeStruct((_NW * CAP * EMBED_DIM,), jnp.float32),
            jax.ShapeDtypeStruct((_NW * CAP * EMBED_DIM,), jnp.float32)),
        scratch_types=[
            pltpu.VMEM((CAPP,), jnp.int32),
            pltpu.VMEM((CAPP,), jnp.int32),
            pltpu.VMEM((CAPP,), jnp.int32),
            pltpu.VMEM((CAPP,), jnp.int32),
            pltpu.VMEM((2, EMBED_DIM, LANES), jnp.float32),
            pltpu.VMEM((2, EMBED_DIM, LANES), jnp.float32),
            pltpu.VMEM((CAP * EMBED_DIM,), jnp.float32),
            pltpu.VMEM((CAP * EMBED_DIM,), jnp.float32),
            pltpu.SemaphoreType.DMA((2,)),
            pltpu.SemaphoreType.DMA((2,)),
        ],
        compiler_params=params,
    )(_extract_body)
    utailT = jnp.pad(utabT[:, TAIL_OFF:], ((0, 0), (0, LANES - TAIL_W)))
    mtailT = jnp.pad(mtabT[:, TAIL_OFF:], ((0, 0), (0, LANES - TAIL_W)))
    interu_flat, interm_flat = extract(utabT, mtabT, utailT, mtailT,
                                       ulane, uch, mlane, mch)
    inter_u = interu_flat.reshape(_IROWS, 2 * EMBED_DIM)
    inter_m = interm_flat.reshape(_IROWS, 2 * EMBED_DIM)

    dot = functools.partial(
        pl.kernel,
        mesh=mesh,
        out_type=jax.ShapeDtypeStruct((BATCH,), jnp.float32),
        scratch_types=[
            pltpu.VMEM((_BPW,), jnp.int32),
            pltpu.VMEM((_BPW,), jnp.int32),
            pltpu.VMEM((_RPP, 2 * EMBED_DIM), jnp.float32),
            pltpu.VMEM((_RPP, 2 * EMBED_DIM), jnp.float32),
            pltpu.VMEM((_RPP,), jnp.int32),
            pltpu.VMEM((_RPP,), jnp.int32),
            pltpu.VMEM((_BPW,), jnp.float32),
            pltpu.SemaphoreType.DMA,
            pltpu.SemaphoreType.DMA,
        ],
        compiler_params=pltpu.CompilerParams(needs_layout_passes=False),
    )(_dot_body)
    return dot(urank, mrank, inter_u, inter_m)


# speedup vs baseline: 1.7752x; 1.7752x over previous
import functools

import jax
import jax.numpy as jnp
from jax import lax
from jax.experimental import pallas as pl
from jax.experimental.pallas import tpu as pltpu
from jax.experimental.pallas import tpu_sc as plsc

BATCH = 16384
EMBED_DIM = 64
NROWS = 1000000
LANES = 128
NCHUNK = 7813
TAIL_C = 7812
TAIL_OFF = TAIL_C * LANES
TAIL_W = NROWS - TAIL_OFF
CAP = 640
MAXCH = 245
MAXCHP = 248
CAPP = CAP + 16

_info = plsc.get_sparse_core_info()
_NC, _NS, _L = _info.num_cores, _info.num_subcores, _info.num_lanes
_NW = _NC * _NS
_BPW = BATCH // _NW
_IROWS = _NW * CAP // 2


def _extract_body(utabT_hbm, mtabT_hbm, utailT_hbm, mtailT_hbm,
                  ulane_hbm, uch_hbm, mlane_hbm, mch_hbm,
                  interu_hbm, interm_hbm,
                  ulane_v, uch_v, mlane_v, mch_v,
                  ubuf, mbuf, stage_u, stage_m, sem_u, sem_m):
    w = lax.axis_index("s") * _NC + lax.axis_index("c")
    ch0 = w * MAXCH
    nch = jnp.minimum(NCHUNK - ch0, MAXCH)

    pltpu.sync_copy(ulane_hbm.at[pl.ds(w * CAPP, CAPP)], ulane_v)
    pltpu.sync_copy(uch_hbm.at[pl.ds(w * CAPP, CAPP)], uch_v)
    pltpu.sync_copy(mlane_hbm.at[pl.ds(w * CAPP, CAPP)], mlane_v)
    pltpu.sync_copy(mch_hbm.at[pl.ds(w * CAPP, CAPP)], mch_v)

    def issue(ci, s):
        c = ch0 + ci

        @pl.when(c < TAIL_C)
        def _():
            pltpu.async_copy(utabT_hbm.at[:, pl.ds(c * LANES, LANES)],
                             ubuf.at[s], sem_u.at[s])
            pltpu.async_copy(mtabT_hbm.at[:, pl.ds(c * LANES, LANES)],
                             mbuf.at[s], sem_m.at[s])

        @pl.when(c == TAIL_C)
        def _():
            pltpu.async_copy(utailT_hbm, ubuf.at[s], sem_u.at[s])
            pltpu.async_copy(mtailT_hbm, mbuf.at[s], sem_m.at[s])

    def wait(s):
        pltpu.make_async_copy(utabT_hbm.at[:, pl.ds(0, LANES)],
                              ubuf.at[s], sem_u.at[s]).wait()
        pltpu.make_async_copy(mtabT_hbm.at[:, pl.ds(0, LANES)],
                              mbuf.at[s], sem_m.at[s]).wait()

    iota = lax.iota(jnp.int32, _L)

    def extract(buf_slot, lane_v, ch_v, stage, p, c):
        def do_window(p0):
            lanes = plsc.load_gather(lane_v, [p0 + iota])
            chs = plsc.load_gather(ch_v, [p0 + iota])
            mask = chs == c
            cnt = plsc.all_reduce_population_count(mask)[0]
            slots = p0 + iota
            base = lax.shift_right_logical(slots, 1) * (2 * EMBED_DIM) \
                + (slots & 1) * EMBED_DIM
            for d in range(EMBED_DIM):
                vals = plsc.load_gather(
                    buf_slot, [jnp.full((_L,), d, jnp.int32), lanes])
                plsc.store_scatter(stage, [base + d], vals, mask=mask)
            return cnt

        cnt0 = do_window(p)
        p1 = p + cnt0

        @pl.when(cnt0 >= _L)
        def _():
            do_window(p1)

        chs1 = plsc.load_gather(ch_v, [p1 + iota])
        cnt1 = plsc.all_reduce_population_count(chs1 == c)[0]
        return p1 + jnp.where(cnt0 >= _L, cnt1, 0)

    issue(0, 0)

    def group_body(g, carry):
        p_u, p_m = carry
        for s in range(2):
            ci = g * 2 + s
            c = ch0 + ci

            @pl.when(ci < nch)
            def _():
                wait(s)

            @pl.when(ci + 1 < nch)
            def _():
                issue(ci + 1, 1 - s)

            p_u = extract(ubuf.at[s], ulane_v, uch_v, stage_u, p_u, c)
            p_m = extract(mbuf.at[s], mlane_v, mch_v, stage_m, p_m, c)
        return (p_u, p_m)

    lax.fori_loop(0, (MAXCH + 1) // 2, group_body,
                  (jnp.int32(0), jnp.int32(0)))

    pltpu.sync_copy(stage_u,
                    interu_hbm.at[pl.ds(w * (CAP * EMBED_DIM),
                                        CAP * EMBED_DIM)])
    pltpu.sync_copy(stage_m,
                    interm_hbm.at[pl.ds(w * (CAP * EMBED_DIM),
                                        CAP * EMBED_DIM)])


_PASS = 2
_RPP = _BPW // _PASS
_CPP = _RPP // _L


def _dot_body(urank_hbm, mrank_hbm, interu_hbm, interm_hbm, out_hbm,
              urank_v, mrank_v, ublk_v, mblk_v, ubi_v, mbi_v, out_v,
              sem_u, sem_m):
    wid = lax.axis_index("s") * _NC + lax.axis_index("c")
    base = wid * _BPW

    pltpu.sync_copy(urank_hbm.at[pl.ds(base, _BPW)], urank_v)
    pltpu.sync_copy(mrank_hbm.at[pl.ds(base, _BPW)], mrank_v)

    def do_pass(p, carry):
        def bi_body(i, c2):
            s = i * _L
            u = urank_v[pl.ds(p * _RPP + s, _L)]
            m = mrank_v[pl.ds(p * _RPP + s, _L)]
            ubi_v[pl.ds(s, _L)] = lax.shift_right_logical(u, 1)
            mbi_v[pl.ds(s, _L)] = lax.shift_right_logical(m, 1)
            return c2

        lax.fori_loop(0, _CPP, bi_body, 0)

        cu = pltpu.async_copy(interu_hbm.at[ubi_v], ublk_v, sem_u)
        cm = pltpu.async_copy(interm_hbm.at[mbi_v], mblk_v, sem_m)
        cu.wait()
        cm.wait()

        def chunk_body(c, c2):
            row_idx = c * _L + lax.iota(jnp.int32, _L)
            uo = urank_v[pl.ds(p * _RPP + c * _L, _L)]
            mo = mrank_v[pl.ds(p * _RPP + c * _L, _L)]
            ucol0 = (uo & 1) * EMBED_DIM
            mcol0 = (mo & 1) * EMBED_DIM
            acc = jnp.zeros((_L,), jnp.float32)
            for d in range(EMBED_DIM):
                u = plsc.load_gather(ublk_v, [row_idx, ucol0 + d])
                m = plsc.load_gather(mblk_v, [row_idx, mcol0 + d])
                acc = acc + u * m
            out_v[pl.ds(p * _RPP + c * _L, _L)] = acc
            return c2

        lax.fori_loop(0, _CPP, chunk_body, 0)
        return carry

    lax.fori_loop(0, _PASS, do_pass, 0)
    pltpu.sync_copy(out_v, out_hbm.at[pl.ds(base, _BPW)])


def _route(r):
    n = r.shape[0]
    order = jnp.argsort(r)
    rs = r[order]
    chunk = lax.shift_right_logical(rs, 7)
    w_k = chunk // MAXCH
    counts = jnp.zeros((_NW,), jnp.int32).at[w_k].add(1)
    st = jnp.concatenate([jnp.zeros((1,), jnp.int32),
                          jnp.cumsum(counts)[:-1].astype(jnp.int32)])
    pos = jnp.arange(n, dtype=jnp.int32) - st[w_k]
    lanes = jnp.zeros((_NW, CAPP), jnp.int32).at[w_k, pos].set(
        rs & (LANES - 1), mode="drop")
    chs = jnp.full((_NW, CAPP), -1, jnp.int32).at[w_k, pos].set(
        chunk, mode="drop")
    rank = jnp.zeros((n,), jnp.int32).at[order].set(w_k * CAP + pos)
    return lanes.reshape(-1), chs.reshape(-1), rank


def kernel(user_indices, movie_indices, user_table, movie_table):
    uidx = user_indices.astype(jnp.int32)
    midx = movie_indices.astype(jnp.int32)
    utabT = user_table.T
    mtabT = movie_table.T
    ulane, uch, urank = _route(uidx)
    mlane, mch, mrank = _route(midx)

    mesh = plsc.VectorSubcoreMesh(core_axis_name="c", subcore_axis_name="s")
    params = pltpu.CompilerParams(
        needs_layout_passes=False, use_tc_tiling_on_sc=True)

    extract = functools.partial(
        pl.kernel,
        mesh=mesh,
        out_type=(
            jax.ShapeDtypeStruct((_NW * CAP * EMBED_DIM,), jnp.float32),
            jax.ShapeDtypeStruct((_NW * CAP * EMBED_DIM,), jnp.float32)),
        scratch_types=[
            pltpu.VMEM((CAPP,), jnp.int32),
            pltpu.VMEM((CAPP,), jnp.int32),
            pltpu.VMEM((CAPP,), jnp.int32),
            pltpu.VMEM((CAPP,), jnp.int32),
            pltpu.VMEM((2, EMBED_DIM, LANES), jnp.float32),
            pltpu.VMEM((2, EMBED_DIM, LANES), jnp.float32),
            pltpu.VMEM((CAP * EMBED_DIM,), jnp.float32),
            pltpu.VMEM((CAP * EMBED_DIM,), jnp.float32),
            pltpu.SemaphoreType.DMA((2,)),
            pltpu.SemaphoreType.DMA((2,)),
        ],
        compiler_params=params,
    )(_extract_body)
    utailT = jnp.pad(utabT[:, TAIL_OFF:], ((0, 0), (0, LANES - TAIL_W)))
    mtailT = jnp.pad(mtabT[:, TAIL_OFF:], ((0, 0), (0, LANES - TAIL_W)))
    interu_flat, interm_flat = extract(utabT, mtabT, utailT, mtailT,
                                       ulane, uch, mlane, mch)
    inter_u = interu_flat.reshape(_IROWS, 2 * EMBED_DIM)
    inter_m = interm_flat.reshape(_IROWS, 2 * EMBED_DIM)

    dot = functools.partial(
        pl.kernel,
        mesh=mesh,
        out_type=jax.ShapeDtypeStruct((BATCH,), jnp.float32),
        scratch_types=[
            pltpu.VMEM((_BPW,), jnp.int32),
            pltpu.VMEM((_BPW,), jnp.int32),
            pltpu.VMEM((_RPP, 2 * EMBED_DIM), jnp.float32),
            pltpu.VMEM((_RPP, 2 * EMBED_DIM), jnp.float32),
            pltpu.VMEM((_RPP,), jnp.int32),
            pltpu.VMEM((_RPP,), jnp.int32),
            pltpu.VMEM((_BPW,), jnp.float32),
            pltpu.SemaphoreType.DMA,
            pltpu.SemaphoreType.DMA,
        ],
        compiler_params=pltpu.CompilerParams(needs_layout_passes=False),
    )(_dot_body)
    return dot(urank, mrank, inter_u, inter_m)


# speedup vs baseline: 2.4253x; 1.3662x over previous
import functools

import jax
import jax.numpy as jnp
from jax import lax
from jax.experimental import pallas as pl
from jax.experimental.pallas import tpu as pltpu
from jax.experimental.pallas import tpu_sc as plsc

BATCH = 16384
EMBED_DIM = 64
NROWS = 1000000
LANES = 512
NCHUNK = 1954
TAIL_C = 1953
TAIL_OFF = TAIL_C * LANES
TAIL_W = NROWS - TAIL_OFF
CAP = 640
MAXCH = 62
CAPP = CAP + 16

_info = plsc.get_sparse_core_info()
_NC, _NS, _L = _info.num_cores, _info.num_subcores, _info.num_lanes
_NW = _NC * _NS
_BPW = BATCH // _NW
_IROWS = _NW * CAP // 2


def _make_extract_body(toff):
  def _extract_body(tabT_hbm, tailT_hbm, ent_hbm, inter_hbm,
                    ent_v, buf, stage, sem):
    w = lax.axis_index("s") * _NC + lax.axis_index("c")
    ch0 = w * MAXCH
    nch = jnp.minimum(NCHUNK - ch0, MAXCH)

    pltpu.sync_copy(ent_hbm.at[pl.ds(toff + w * CAPP, CAPP)], ent_v)

    def issue(ci, s):
        c = ch0 + ci

        @pl.when(c < TAIL_C)
        def _():
            pltpu.async_copy(tabT_hbm.at[:, pl.ds(c * LANES, LANES)],
                             buf.at[s], sem.at[s])

        @pl.when(c == TAIL_C)
        def _():
            pltpu.async_copy(tailT_hbm, buf.at[s], sem.at[s])

    def wait(s):
        pltpu.make_async_copy(tabT_hbm.at[:, pl.ds(0, LANES)],
                              buf.at[s], sem.at[s]).wait()

    iota = lax.iota(jnp.int32, _L)

    def extract(buf_slot, p, c):
        def window(p0):
            ent = plsc.load_gather(ent_v, [p0 + iota])
            lanes = ent & (LANES - 1)
            chs = lax.shift_right_arithmetic(ent, 9)
            mask = chs == c
            cnt = plsc.all_reduce_population_count(mask)[0]
            slots = p0 + iota
            base = lax.shift_right_logical(slots, 1) * (2 * EMBED_DIM) \
                + (slots & 1) * EMBED_DIM
            for d in range(EMBED_DIM):
                vals = plsc.load_gather(
                    buf_slot, [jnp.full((_L,), d, jnp.int32), lanes])
                plsc.store_scatter(stage, [base + d], vals, mask=mask)
            return cnt

        cnt0 = window(p)
        p1 = p + cnt0
        full0 = cnt0 >= _L

        @pl.when(full0)
        def _():
            window(p1)

        ch1 = lax.shift_right_arithmetic(
            plsc.load_gather(ent_v, [p1 + iota]), 9)
        cnt1 = plsc.all_reduce_population_count(ch1 == c)[0]
        p2 = p1 + jnp.where(full0, cnt1, 0)
        full1 = full0 & (cnt1 >= _L)

        @pl.when(full1)
        def _():
            window(p2)

        ch2 = lax.shift_right_arithmetic(
            plsc.load_gather(ent_v, [jnp.minimum(p2 + iota, CAPP - 1)]), 9)
        cnt2 = plsc.all_reduce_population_count(ch2 == c)[0]
        return p2 + jnp.where(full1, cnt2, 0)

    issue(0, 0)

    def group_body(g, p):
        for s in range(2):
            ci = g * 2 + s
            c = ch0 + ci

            @pl.when(ci < nch)
            def _():
                wait(s)

            @pl.when(ci + 1 < nch)
            def _():
                issue(ci + 1, 1 - s)

            p = extract(buf.at[s], p, c)
        return p

    lax.fori_loop(0, (MAXCH + 1) // 2, group_body, jnp.int32(0))

    pltpu.sync_copy(stage,
                    inter_hbm.at[pl.ds(w * (CAP * EMBED_DIM),
                                       CAP * EMBED_DIM)])

  return _extract_body


_PASS = 2
_RPP = _BPW // _PASS
_CPP = _RPP // _L


def _dot_body(urank_hbm, mrank_hbm, interu_hbm, interm_hbm, out_hbm,
              urank_v, mrank_v, ublk_v, mblk_v, ubi_v, mbi_v, out_v,
              sem_u, sem_m):
    wid = lax.axis_index("s") * _NC + lax.axis_index("c")
    base = wid * _BPW

    pltpu.sync_copy(urank_hbm.at[pl.ds(base, _BPW)], urank_v)
    pltpu.sync_copy(mrank_hbm.at[pl.ds(base, _BPW)], mrank_v)

    def do_pass(p, carry):
        def bi_body(i, c2):
            s = i * _L
            u = urank_v[pl.ds(p * _RPP + s, _L)]
            m = mrank_v[pl.ds(p * _RPP + s, _L)]
            ubi_v[pl.ds(s, _L)] = lax.shift_right_logical(u, 1)
            mbi_v[pl.ds(s, _L)] = lax.shift_right_logical(m, 1)
            return c2

        lax.fori_loop(0, _CPP, bi_body, 0)

        cu = pltpu.async_copy(interu_hbm.at[ubi_v], ublk_v, sem_u)
        cm = pltpu.async_copy(interm_hbm.at[mbi_v], mblk_v, sem_m)
        cu.wait()
        cm.wait()

        def chunk_body(c, c2):
            row_idx = c * _L + lax.iota(jnp.int32, _L)
            uo = urank_v[pl.ds(p * _RPP + c * _L, _L)]
            mo = mrank_v[pl.ds(p * _RPP + c * _L, _L)]
            ucol0 = (uo & 1) * EMBED_DIM
            mcol0 = (mo & 1) * EMBED_DIM
            acc = jnp.zeros((_L,), jnp.float32)
            for d in range(EMBED_DIM):
                u = plsc.load_gather(ublk_v, [row_idx, ucol0 + d])
                m = plsc.load_gather(mblk_v, [row_idx, mcol0 + d])
                acc = acc + u * m
            out_v[pl.ds(p * _RPP + c * _L, _L)] = acc
            return c2

        lax.fori_loop(0, _CPP, chunk_body, 0)
        return carry

    lax.fori_loop(0, _PASS, do_pass, 0)
    pltpu.sync_copy(out_v, out_hbm.at[pl.ds(base, _BPW)])


def _route(r):
    n = r.shape[0]
    order = jnp.argsort(r)
    rs = r[order]
    w_k = lax.shift_right_logical(rs, 9) // MAXCH
    counts = (w_k[:, None] == jnp.arange(_NW, dtype=jnp.int32)[None, :]
              ).sum(axis=0).astype(jnp.int32)
    st = jnp.concatenate([jnp.zeros((1,), jnp.int32),
                          jnp.cumsum(counts)[:-1].astype(jnp.int32)])
    pos = jnp.arange(n, dtype=jnp.int32) - st[w_k]
    j = jnp.arange(_NW * CAPP, dtype=jnp.int32)
    w_j = j // CAPP
    p_j = j - w_j * CAPP
    k_j = jnp.clip(st[w_j] + p_j, 0, n - 1)
    ent = jnp.where(p_j < counts[w_j], rs[k_j], -1)
    inv = jnp.argsort(order)
    rank = (w_k * CAP + pos)[inv]
    return ent, rank


def kernel(user_indices, movie_indices, user_table, movie_table):
    uidx = user_indices.astype(jnp.int32)
    midx = movie_indices.astype(jnp.int32)
    utabT = user_table.T
    mtabT = movie_table.T
    uent, urank = _route(uidx)
    ment, mrank = _route(midx)

    mesh = plsc.VectorSubcoreMesh(core_axis_name="c", subcore_axis_name="s")
    params = pltpu.CompilerParams(
        needs_layout_passes=False, use_tc_tiling_on_sc=True)

    extract = functools.partial(
        pl.kernel,
        mesh=mesh,
        out_type=jax.ShapeDtypeStruct((_NW * CAP * EMBED_DIM,), jnp.float32),
        scratch_types=[
            pltpu.VMEM((CAPP,), jnp.int32),
            pltpu.VMEM((2, EMBED_DIM, LANES), jnp.float32),
            pltpu.VMEM((CAP * EMBED_DIM,), jnp.float32),
            pltpu.SemaphoreType.DMA((2,)),
        ],
        compiler_params=params,
    )(_make_extract_body(0))
    utailT = jnp.pad(utabT[:, TAIL_OFF:], ((0, 0), (0, LANES - TAIL_W)))
    mtailT = jnp.pad(mtabT[:, TAIL_OFF:], ((0, 0), (0, LANES - TAIL_W)))
    inter_u = extract(utabT, utailT, uent).reshape(_IROWS, 2 * EMBED_DIM)
    inter_m = extract(mtabT, mtailT, ment).reshape(_IROWS, 2 * EMBED_DIM)

    dot = functools.partial(
        pl.kernel,
        mesh=mesh,
        out_type=jax.ShapeDtypeStruct((BATCH,), jnp.float32),
        scratch_types=[
            pltpu.VMEM((_BPW,), jnp.int32),
            pltpu.VMEM((_BPW,), jnp.int32),
            pltpu.VMEM((_RPP, 2 * EMBED_DIM), jnp.float32),
            pltpu.VMEM((_RPP, 2 * EMBED_DIM), jnp.float32),
            pltpu.VMEM((_RPP,), jnp.int32),
            pltpu.VMEM((_RPP,), jnp.int32),
            pltpu.VMEM((_BPW,), jnp.float32),
            pltpu.SemaphoreType.DMA,
            pltpu.SemaphoreType.DMA,
        ],
        compiler_params=pltpu.CompilerParams(needs_layout_passes=False),
    )(_dot_body)
    return dot(urank, mrank, inter_u, inter_m)
